# baseline (speedup 1.0000x reference)
import functools

import jax
import jax.numpy as jnp
from jax import lax
from jax.experimental import pallas as pl
from jax.experimental.pallas import tpu as pltpu
from jax.experimental.pallas import tpu_sc as plsc

N = 4096
CH = 64
N_LAYERS = 2


def _topk_call(x, k, br=512):
    n, f = x.shape

    def body(xb_ref, xf_ref, idx_ref):
        i = pl.program_id(0)
        xb = xb_ref[...]
        xf = xf_ref[...]
        sqb = jnp.sum(xb * xb, axis=1, keepdims=True)
        ones = jnp.ones((1, f), jnp.float32)
        sqr = lax.dot_general(ones, xf * xf, (((1,), (1,)), ((), ())),
                              preferred_element_type=jnp.float32,
                              precision=lax.Precision.HIGHEST)
        p = lax.dot_general(xb.astype(jnp.bfloat16), xf.astype(jnp.bfloat16),
                            (((1,), (1,)), ((), ())),
                            preferred_element_type=jnp.float32)
        cols = lax.broadcasted_iota(jnp.int32, (br, n), 1)
        rows = lax.broadcasted_iota(jnp.int32, (br, n), 0) + i * br
        d = (sqb - 2.0 * p) + sqr
        d = d + jnp.where(cols == rows, jnp.float32(1e10), jnp.float32(0.0))
        kcols = lax.broadcasted_iota(jnp.int32, (br, k), 1)
        acc = jnp.zeros((br, k), jnp.int32)
        for t in range(k):
            a = jnp.argmin(d, axis=1).astype(jnp.int32)
            acc = jnp.where(kcols == t, a[:, None], acc)
            d = jnp.where(cols == a[:, None], jnp.float32(jnp.inf), d)
        idx_ref[...] = acc

    return pl.pallas_call(
        body,
        grid=(n // br,),
        in_specs=[
            pl.BlockSpec((br, f), lambda i: (i, 0)),
            pl.BlockSpec((n, f), lambda i: (0, 0)),
        ],
        out_specs=pl.BlockSpec((br, k), lambda i: (i, 0)),
        out_shape=jax.ShapeDtypeStruct((n, k), jnp.int32),
    )(x, x)


def _mm_call(x, wa, wb, b2, br=512):
    n, c = x.shape
    co = wa.shape[1]

    def body(x_ref, wa_ref, wb_ref, b_ref, o_ref):
        x = x_ref[...]
        qa = jnp.dot(x.astype(jnp.bfloat16), wa_ref[...].astype(jnp.bfloat16),
                     preferred_element_type=jnp.float32)
        wb = wb_ref[...].astype(jnp.bfloat16).astype(jnp.float32)
        qb = jnp.dot(x, wb, preferred_element_type=jnp.float32,
                     precision=lax.Precision.HIGHEST)
        o_ref[...] = qa + qb + b_ref[...]

    return pl.pallas_call(
        body,
        grid=(n // br,),
        in_specs=[
            pl.BlockSpec((br, c), lambda i: (i, 0)),
            pl.BlockSpec((c, co), lambda i: (0, 0)),
            pl.BlockSpec((c, co), lambda i: (0, 0)),
            pl.BlockSpec((1, co), lambda i: (0, 0)),
        ],
        out_specs=pl.BlockSpec((br, co), lambda i: (i, 0)),
        out_shape=jax.ShapeDtypeStruct((n, co), jnp.float32),
    )(x, wa, wb, b2)


def _make_gather_max(k):
    nc, ns = 2, 16
    nw = nc * ns
    rows_w = N // nw
    rpc = 128 // k
    nchunks = rows_w // rpc

    @functools.partial(
        pl.kernel,
        mesh=plsc.VectorSubcoreMesh(core_axis_name="c", subcore_axis_name="s"),
        out_type=jax.ShapeDtypeStruct((N, CH), jnp.float32),
        scratch_types=[
            pltpu.VMEM((128,), jnp.int32),
            pltpu.VMEM((128, 2 * CH), jnp.float32),
            pltpu.VMEM((rpc, 2 * CH), jnp.float32),
            pltpu.VMEM((rpc, CH), jnp.float32),
            pltpu.SemaphoreType.DMA,
        ],
    )
    def gather_max(cb_hbm, idx_hbm, out_hbm, idx_v, rows_v, c_v, out_v, sem):
        wid = lax.axis_index("s") * nc + lax.axis_index("c")
        row0 = wid * rows_w

        def chunk(ci, carry):
            rbase = row0 + ci * rpc
            pltpu.sync_copy(idx_hbm.at[pl.ds(rbase * k, 128)], idx_v)
            pltpu.async_copy(cb_hbm.at[idx_v], rows_v, sem).wait()
            pltpu.sync_copy(cb_hbm.at[pl.ds(rbase, rpc)], c_v)
            for r in range(rpc):
                for blk in range(CH // 16):
                    msl = pl.ds(CH + blk * 16, 16)
                    acc = rows_v[r * k, msl]
                    for j in range(1, k):
                        acc = jnp.maximum(acc, rows_v[r * k + j, msl])
                    out_v[r, pl.ds(blk * 16, 16)] = jnp.maximum(
                        acc + c_v[r, pl.ds(blk * 16, 16)], jnp.float32(0.0))
            pltpu.sync_copy(out_v, out_hbm.at[pl.ds(rbase, rpc)])
            return carry

        lax.fori_loop(0, nchunks, chunk, 0)

    return gather_max


def _make_gather_pipe():
    nw = 32
    nr = 2 * N
    rows_w = nr // nw
    rpc = 8
    nch = rows_w // rpc

    @functools.partial(
        pl.kernel,
        mesh=plsc.VectorSubcoreMesh(core_axis_name="c", subcore_axis_name="s"),
        out_type=jax.ShapeDtypeStruct((nr, CH), jnp.float32),
        scratch_types=[
            pltpu.VMEM((nch, 128), jnp.int32),
            pltpu.VMEM((rows_w, 2 * CH), jnp.float32),
            pltpu.VMEM((2, 128, 2 * CH), jnp.float32),
            pltpu.VMEM((rows_w, CH), jnp.float32),
            pltpu.SemaphoreType.DMA,
            pltpu.SemaphoreType.DMA,
        ],
    )
    def gather_pipe(cb_hbm, idx_hbm, out_hbm, idx_all, c_all, rows_v, out_all,
                    sem0, sem1):
        wid = lax.axis_index("s") * 2 + lax.axis_index("c")
        row0 = wid * rows_w
        pltpu.sync_copy(idx_hbm.at[pl.ds(wid * nch, nch)], idx_all)
        pltpu.sync_copy(cb_hbm.at[pl.ds(row0, rows_w)], c_all)
        pltpu.async_copy(cb_hbm.at[idx_all.at[0]], rows_v.at[0], sem0)

        def compute_chunk(q, b):
            for r in range(rpc):
                row = q * rpc + r
                for blk in range(CH // 16):
                    msl = pl.ds(CH + blk * 16, 16)
                    acc = rows_v[b, r * 16, msl]
                    for j in range(1, 16):
                        acc = jnp.maximum(acc, rows_v[b, r * 16 + j, msl])
                    sl = pl.ds(blk * 16, 16)
                    out_all[row, sl] = jnp.maximum(acc + c_all[row, sl],
                                                   jnp.float32(0.0))

        def it(i, carry):
            pltpu.async_copy(cb_hbm.at[idx_all.at[2 * i + 1]], rows_v.at[1],
                             sem1)
            pltpu.make_async_copy(cb_hbm.at[idx_all.at[0]], rows_v.at[0],
                                  sem0).wait()
            compute_chunk(2 * i, 0)

            @pl.when(i < nch // 2 - 1)
            def _():
                pltpu.async_copy(cb_hbm.at[idx_all.at[2 * i + 2]],
                                 rows_v.at[0], sem0)

            pltpu.make_async_copy(cb_hbm.at[idx_all.at[0]], rows_v.at[1],
                                  sem1).wait()
            compute_chunk(2 * i + 1, 1)
            return carry

        lax.fori_loop(0, nch // 2, it, 0)
        pltpu.sync_copy(out_all, out_hbm.at[pl.ds(row0, rows_w)])

    return gather_pipe


_GM_CACHE = {}


def _gather_max_fn(k):
    if k not in _GM_CACHE:
        _GM_CACHE[k] = _make_gather_max(k)
    return _GM_CACHE[k]


def _gather_pipe_fn():
    if "pipe" not in _GM_CACHE:
        _GM_CACHE["pipe"] = _make_gather_pipe()
    return _GM_CACHE["pipe"]


def _combine_call(g1, g2, x, res_in, scale, br=512):
    def body(a0, a1, a2, a3, b0, b1, b2, b3, x_ref, r_ref, xo_ref, ro_ref):
        h1 = jnp.maximum(jnp.maximum(a0[...], a1[...]),
                         jnp.maximum(a2[...], a3[...]))
        h2 = jnp.maximum(jnp.maximum(b0[...], b1[...]),
                         jnp.maximum(b2[...], b3[...]))
        xn = jnp.maximum(h1, h2) + x_ref[...]
        xo_ref[...] = xn
        ro_ref[...] = (r_ref[...] + xn) * jnp.float32(scale)

    spec = pl.BlockSpec((br, CH), lambda i: (i, 0))
    return pl.pallas_call(
        body,
        grid=(N // br,),
        in_specs=[spec] * 10,
        out_specs=[spec, spec],
        out_shape=[
            jax.ShapeDtypeStruct((N, CH), jnp.float32),
            jax.ShapeDtypeStruct((N, CH), jnp.float32),
        ],
    )(*g1, *g2, x, res_in)


def _edge_weights(wt, wb, bvec):
    z = jnp.zeros_like(wt)
    wa = jnp.concatenate([wt, z], axis=1)
    wb2 = jnp.concatenate([-wb, wb], axis=1)
    b2 = jnp.concatenate([bvec, jnp.zeros_like(bvec)])[None, :]
    return wa, wb2, b2


def _edge_cb(xc, w, bvec):
    ic = w.shape[0] // 2
    wa, wb2, b2 = _edge_weights(w[:ic], w[ic:], bvec)
    return _mm_call(xc, wa, wb2, b2)


def _mm_call2(xs, wa_s, wb_s, b_s, br=512):
    n2, c = xs.shape
    co = wa_s.shape[2]

    def body(x_ref, wa_ref, wb_ref, b_ref, o_ref):
        x = x_ref[...]
        qa = jnp.dot(x.astype(jnp.bfloat16),
                     wa_ref[0].astype(jnp.bfloat16),
                     preferred_element_type=jnp.float32)
        wb = wb_ref[0].astype(jnp.bfloat16).astype(jnp.float32)
        qb = jnp.dot(x, wb, preferred_element_type=jnp.float32,
                     precision=lax.Precision.HIGHEST)
        o_ref[...] = qa + qb + b_ref[0]

    half = N // br

    return pl.pallas_call(
        body,
        grid=(n2 // br,),
        in_specs=[
            pl.BlockSpec((br, c), lambda i: (i, 0)),
            pl.BlockSpec((1, c, co), lambda i: (i // half, 0, 0)),
            pl.BlockSpec((1, c, co), lambda i: (i // half, 0, 0)),
            pl.BlockSpec((1, 1, co), lambda i: (i // half, 0, 0)),
        ],
        out_specs=pl.BlockSpec((br, co), lambda i: (i, 0)),
        out_shape=jax.ShapeDtypeStruct((n2, co), jnp.float32),
    )(xs, wa_s, wb_s, b_s)


def kernel(x, params):
    xp = jnp.pad(x, ((0, 0), (0, 5)))
    idx1 = _topk_call(xp, 1)
    w_top = jnp.pad(params["pre_W"][:3], ((0, 5), (0, 0)))
    w_bot = jnp.pad(params["pre_W"][3:], ((0, 5), (0, 0)))
    wa, wb2, b2 = _edge_weights(w_top, w_bot, params["pre_b"])
    cb = _mm_call(xp, wa, wb2, b2)
    xf = _gather_max_fn(1)(cb, idx1.reshape(-1))

    res = jnp.zeros((N, CH), jnp.float32)
    for li, lp in enumerate(params["layers"]):
        idx32 = _topk_call(xf, 32)
        idx_a = idx32[:, :16].reshape(-1)
        idx_b = idx32[:, ::2].reshape(-1)
        idxb = jnp.concatenate([idx_a, idx_b + N]).reshape(-1, 128)
        xc1 = xc2 = xf
        for bi in range(len(lp["gcn1"])):
            w1, b1 = lp["gcn1"][bi]
            w2r, b2r = lp["gcn2"][bi]
            ic = w1.shape[0] // 2
            wa1, wb1, bb1 = _edge_weights(w1[:ic], w1[ic:], b1)
            wa2, wb2, bb2 = _edge_weights(w2r[:ic], w2r[ic:], b2r)
            wa_s = jnp.stack([wa1, wa2])
            wb_s = jnp.stack([wb1, wb2])
            b_s = jnp.stack([bb1, bb2])
            xs = jnp.concatenate([xc1, xc2], axis=0)
            cbb = _mm_call2(xs, wa_s, wb_s, b_s)
            ob = _gather_pipe_fn()(cbb, idxb)
            xc1 = jnp.concatenate([xc1, ob[:N]], axis=1)
            xc2 = jnp.concatenate([xc2, ob[N:]], axis=1)
        chains = [xc1, xc2]
        g1 = [chains[0][:, t::4] for t in range(4)]
        g2 = [chains[1][:, t::4] for t in range(4)]
        scale = 1.0 / N_LAYERS if li == N_LAYERS - 1 else 1.0
        xf, res = _combine_call(g1, g2, xf, res, scale)
    return res


# speedup vs baseline: 1.0585x; 1.0585x over previous
import functools

import jax
import jax.numpy as jnp
from jax import lax
from jax.experimental import pallas as pl
from jax.experimental.pallas import tpu as pltpu
from jax.experimental.pallas import tpu_sc as plsc

N = 4096
CH = 64
N_LAYERS = 2


def _topk_call(x, k, br=256):
    n, f = x.shape

    def body(xb_ref, xf_ref, idx_ref):
        i = pl.program_id(0)
        xb = xb_ref[...]
        xf = xf_ref[...]
        sqb = jnp.sum(xb * xb, axis=1, keepdims=True)
        ones = jnp.ones((1, f), jnp.float32)
        sqr = lax.dot_general(ones, xf * xf, (((1,), (1,)), ((), ())),
                              preferred_element_type=jnp.float32,
                              precision=lax.Precision.HIGHEST)
        p = lax.dot_general(xb.astype(jnp.bfloat16), xf.astype(jnp.bfloat16),
                            (((1,), (1,)), ((), ())),
                            preferred_element_type=jnp.float32)
        cols = lax.broadcasted_iota(jnp.int32, (br, n), 1)
        rows = lax.broadcasted_iota(jnp.int32, (br, n), 0) + i * br
        d = (sqb - 2.0 * p) + sqr
        d = d + jnp.where(cols == rows, jnp.float32(1e10), jnp.float32(0.0))
        kcols = lax.broadcasted_iota(jnp.int32, (br, k), 1)
        acc = jnp.zeros((br, k), jnp.int32)
        for t in range(k):
            a = jnp.argmin(d, axis=1).astype(jnp.int32)
            acc = jnp.where(kcols == t, a[:, None], acc)
            d = jnp.where(cols == a[:, None], jnp.float32(jnp.inf), d)
        idx_ref[...] = acc

    return pl.pallas_call(
        body,
        grid=(n // br,),
        in_specs=[
            pl.BlockSpec((br, f), lambda i: (i, 0)),
            pl.BlockSpec((n, f), lambda i: (0, 0)),
        ],
        out_specs=pl.BlockSpec((br, k), lambda i: (i, 0)),
        out_shape=jax.ShapeDtypeStruct((n, k), jnp.int32),
    )(x, x)


def _mm_call(x, wa, wb, b2, br=512):
    n, c = x.shape
    co = wa.shape[1]

    def body(x_ref, wa_ref, wb_ref, b_ref, o_ref):
        x = x_ref[...]
        qa = jnp.dot(x.astype(jnp.bfloat16), wa_ref[...].astype(jnp.bfloat16),
                     preferred_element_type=jnp.float32)
        wb = wb_ref[...].astype(jnp.bfloat16).astype(jnp.float32)
        qb = jnp.dot(x, wb, preferred_element_type=jnp.float32,
                     precision=lax.Precision.HIGHEST)
        o_ref[...] = qa + qb + b_ref[...]

    return pl.pallas_call(
        body,
        grid=(n // br,),
        in_specs=[
            pl.BlockSpec((br, c), lambda i: (i, 0)),
            pl.BlockSpec((c, co), lambda i: (0, 0)),
            pl.BlockSpec((c, co), lambda i: (0, 0)),
            pl.BlockSpec((1, co), lambda i: (0, 0)),
        ],
        out_specs=pl.BlockSpec((br, co), lambda i: (i, 0)),
        out_shape=jax.ShapeDtypeStruct((n, co), jnp.float32),
    )(x, wa, wb, b2)


def _make_gather_max(k):
    nc, ns = 2, 16
    nw = nc * ns
    rows_w = N // nw
    rpc = 128 // k
    nchunks = rows_w // rpc

    @functools.partial(
        pl.kernel,
        mesh=plsc.VectorSubcoreMesh(core_axis_name="c", subcore_axis_name="s"),
        out_type=jax.ShapeDtypeStruct((N, CH), jnp.float32),
        scratch_types=[
            pltpu.VMEM((128,), jnp.int32),
            pltpu.VMEM((128, 2 * CH), jnp.float32),
            pltpu.VMEM((rpc, 2 * CH), jnp.float32),
            pltpu.VMEM((rpc, CH), jnp.float32),
            pltpu.SemaphoreType.DMA,
        ],
    )
    def gather_max(cb_hbm, idx_hbm, out_hbm, idx_v, rows_v, c_v, out_v, sem):
        wid = lax.axis_index("s") * nc + lax.axis_index("c")
        row0 = wid * rows_w

        def chunk(ci, carry):
            rbase = row0 + ci * rpc
            pltpu.sync_copy(idx_hbm.at[pl.ds(rbase * k, 128)], idx_v)
            pltpu.async_copy(cb_hbm.at[idx_v], rows_v, sem).wait()
            pltpu.sync_copy(cb_hbm.at[pl.ds(rbase, rpc)], c_v)
            for r in range(rpc):
                for blk in range(CH // 16):
                    msl = pl.ds(CH + blk * 16, 16)
                    acc = rows_v[r * k, msl]
                    for j in range(1, k):
                        acc = jnp.maximum(acc, rows_v[r * k + j, msl])
                    out_v[r, pl.ds(blk * 16, 16)] = jnp.maximum(
                        acc + c_v[r, pl.ds(blk * 16, 16)], jnp.float32(0.0))
            pltpu.sync_copy(out_v, out_hbm.at[pl.ds(rbase, rpc)])
            return carry

        lax.fori_loop(0, nchunks, chunk, 0)

    return gather_max


def _make_gather_pipe():
    nw = 32
    nr = 2 * N
    rows_w = nr // nw
    rpc = 8
    nch = rows_w // rpc

    @functools.partial(
        pl.kernel,
        mesh=plsc.VectorSubcoreMesh(core_axis_name="c", subcore_axis_name="s"),
        out_type=jax.ShapeDtypeStruct((nr, CH), jnp.float32),
        scratch_types=[
            pltpu.VMEM((nch, 128), jnp.int32),
            pltpu.VMEM((rows_w, 2 * CH), jnp.float32),
            pltpu.VMEM((2, 128, 2 * CH), jnp.float32),
            pltpu.VMEM((rows_w, CH), jnp.float32),
            pltpu.SemaphoreType.DMA,
            pltpu.SemaphoreType.DMA,
        ],
    )
    def gather_pipe(cb_hbm, idx_hbm, out_hbm, idx_all, c_all, rows_v, out_all,
                    sem0, sem1):
        wid = lax.axis_index("s") * 2 + lax.axis_index("c")
        row0 = wid * rows_w
        pltpu.sync_copy(idx_hbm.at[pl.ds(wid * nch, nch)], idx_all)
        pltpu.sync_copy(cb_hbm.at[pl.ds(row0, rows_w)], c_all)
        pltpu.async_copy(cb_hbm.at[idx_all.at[0]], rows_v.at[0], sem0)

        def compute_chunk(q, b):
            for r in range(rpc):
                row = q * rpc + r
                for blk in range(CH // 16):
                    msl = pl.ds(CH + blk * 16, 16)
                    acc = rows_v[b, r * 16, msl]
                    for j in range(1, 16):
                        acc = jnp.maximum(acc, rows_v[b, r * 16 + j, msl])
                    sl = pl.ds(blk * 16, 16)
                    out_all[row, sl] = jnp.maximum(acc + c_all[row, sl],
                                                   jnp.float32(0.0))

        def it(i, carry):
            pltpu.async_copy(cb_hbm.at[idx_all.at[2 * i + 1]], rows_v.at[1],
                             sem1)
            pltpu.make_async_copy(cb_hbm.at[idx_all.at[0]], rows_v.at[0],
                                  sem0).wait()
            compute_chunk(2 * i, 0)

            @pl.when(i < nch // 2 - 1)
            def _():
                pltpu.async_copy(cb_hbm.at[idx_all.at[2 * i + 2]],
                                 rows_v.at[0], sem0)

            pltpu.make_async_copy(cb_hbm.at[idx_all.at[0]], rows_v.at[1],
                                  sem1).wait()
            compute_chunk(2 * i + 1, 1)
            return carry

        lax.fori_loop(0, nch // 2, it, 0)
        pltpu.sync_copy(out_all, out_hbm.at[pl.ds(row0, rows_w)])

    return gather_pipe


_GM_CACHE = {}


def _gather_max_fn(k):
    if k not in _GM_CACHE:
        _GM_CACHE[k] = _make_gather_max(k)
    return _GM_CACHE[k]


def _gather_pipe_fn():
    if "pipe" not in _GM_CACHE:
        _GM_CACHE["pipe"] = _make_gather_pipe()
    return _GM_CACHE["pipe"]


def _combine_call(g1, g2, x, res_in, scale, br=512):
    def body(a0, a1, a2, a3, b0, b1, b2, b3, x_ref, r_ref, xo_ref, ro_ref):
        h1 = jnp.maximum(jnp.maximum(a0[...], a1[...]),
                         jnp.maximum(a2[...], a3[...]))
        h2 = jnp.maximum(jnp.maximum(b0[...], b1[...]),
                         jnp.maximum(b2[...], b3[...]))
        xn = jnp.maximum(h1, h2) + x_ref[...]
        xo_ref[...] = xn
        ro_ref[...] = (r_ref[...] + xn) * jnp.float32(scale)

    spec = pl.BlockSpec((br, CH), lambda i: (i, 0))
    return pl.pallas_call(
        body,
        grid=(N // br,),
        in_specs=[spec] * 10,
        out_specs=[spec, spec],
        out_shape=[
            jax.ShapeDtypeStruct((N, CH), jnp.float32),
            jax.ShapeDtypeStruct((N, CH), jnp.float32),
        ],
    )(*g1, *g2, x, res_in)


def _edge_weights(wt, wb, bvec):
    z = jnp.zeros_like(wt)
    wa = jnp.concatenate([wt, z], axis=1)
    wb2 = jnp.concatenate([-wb, wb], axis=1)
    b2 = jnp.concatenate([bvec, jnp.zeros_like(bvec)])[None, :]
    return wa, wb2, b2


def _edge_cb(xc, w, bvec):
    ic = w.shape[0] // 2
    wa, wb2, b2 = _edge_weights(w[:ic], w[ic:], bvec)
    return _mm_call(xc, wa, wb2, b2)


def _mm_call2(xs, wa_s, wb_s, b_s, br=512):
    n2, c = xs.shape
    co = wa_s.shape[2]

    def body(x_ref, wa_ref, wb_ref, b_ref, o_ref):
        x = x_ref[...]
        qa = jnp.dot(x.astype(jnp.bfloat16),
                     wa_ref[0].astype(jnp.bfloat16),
                     preferred_element_type=jnp.float32)
        wb = wb_ref[0].astype(jnp.bfloat16).astype(jnp.float32)
        qb = jnp.dot(x, wb, preferred_element_type=jnp.float32,
                     precision=lax.Precision.HIGHEST)
        o_ref[...] = qa + qb + b_ref[0]

    half = N // br

    return pl.pallas_call(
        body,
        grid=(n2 // br,),
        in_specs=[
            pl.BlockSpec((br, c), lambda i: (i, 0)),
            pl.BlockSpec((1, c, co), lambda i: (i // half, 0, 0)),
            pl.BlockSpec((1, c, co), lambda i: (i // half, 0, 0)),
            pl.BlockSpec((1, 1, co), lambda i: (i // half, 0, 0)),
        ],
        out_specs=pl.BlockSpec((br, co), lambda i: (i, 0)),
        out_shape=jax.ShapeDtypeStruct((n2, co), jnp.float32),
    )(xs, wa_s, wb_s, b_s)


def kernel(x, params):
    xp = jnp.pad(x, ((0, 0), (0, 5)))
    idx1 = _topk_call(xp, 1)
    w_top = jnp.pad(params["pre_W"][:3], ((0, 5), (0, 0)))
    w_bot = jnp.pad(params["pre_W"][3:], ((0, 5), (0, 0)))
    wa, wb2, b2 = _edge_weights(w_top, w_bot, params["pre_b"])
    cb = _mm_call(xp, wa, wb2, b2)
    xf = _gather_max_fn(1)(cb, idx1.reshape(-1))

    res = jnp.zeros((N, CH), jnp.float32)
    for li, lp in enumerate(params["layers"]):
        idx32 = _topk_call(xf, 32)
        idx_a = idx32[:, :16].reshape(-1)
        idx_b = idx32[:, ::2].reshape(-1)
        idxb = jnp.concatenate([idx_a, idx_b + N]).reshape(-1, 128)
        xc1 = xc2 = xf
        for bi in range(len(lp["gcn1"])):
            w1, b1 = lp["gcn1"][bi]
            w2r, b2r = lp["gcn2"][bi]
            ic = w1.shape[0] // 2
            wa1, wb1, bb1 = _edge_weights(w1[:ic], w1[ic:], b1)
            wa2, wb2, bb2 = _edge_weights(w2r[:ic], w2r[ic:], b2r)
            wa_s = jnp.stack([wa1, wa2])
            wb_s = jnp.stack([wb1, wb2])
            b_s = jnp.stack([bb1, bb2])
            xs = jnp.concatenate([xc1, xc2], axis=0)
            cbb = _mm_call2(xs, wa_s, wb_s, b_s)
            ob = _gather_pipe_fn()(cbb, idxb)
            xc1 = jnp.concatenate([xc1, ob[:N]], axis=1)
            xc2 = jnp.concatenate([xc2, ob[N:]], axis=1)
        chains = [xc1, xc2]
        g1 = [chains[0][:, t::4] for t in range(4)]
        g2 = [chains[1][:, t::4] for t in range(4)]
        scale = 1.0 / N_LAYERS if li == N_LAYERS - 1 else 1.0
        xf, res = _combine_call(g1, g2, xf, res, scale)
    return res
